# baseline (speedup 1.0000x reference)
import jax
import jax.numpy as jnp
from jax import lax
from jax.experimental import pallas as pl
from jax.experimental.pallas import tpu as pltpu
from jax.experimental.pallas import tpu_sc as plsc

E = 1000000
D = 64
B = 16384
NC = 2
NS = 16

NTILE_E = 32768
W = 256
NSLAB = NTILE_E // W
TAIL_E = 999936
FLUSH = 32
GROWS = 48
T_OFF = B + 8
SCR_ROWS = 2 * T_OFF + 16
DUMP = B + 4
SENT = (32767 << 16) | (B + 16390)

CHUNK = 2048
NCHUNK = B // CHUNK


def _bodyA(head_h, tail_h, ent_t_h, tailtab_h, scr_h,
           cb, matchbuf, sorted_, pend, slab, gb, jb0, jb1, st,
           sem_c, sem_s, sem_w):
    iota = lax.iota(jnp.int32, 16)
    wid = lax.axis_index("s") * NC + lax.axis_index("c")
    jbs = (jb0, jb1)

    @pl.when(wid <= 30)
    def _phase_a():
        for k in range(8):
            st[k] = 0

        def reset_jb(ref):
            ref[pl.ds(0, 16)] = jnp.full((16,), DUMP, jnp.int32)
            ref[pl.ds(16, 16)] = jnp.full((16,), DUMP, jnp.int32)
            ref[pl.ds(32, 16)] = jnp.full((16,), DUMP, jnp.int32)

        for q in range(2):
            reset_jb(jbs[q])

        def fire_chunk(p, k):
            if k < NCHUNK:
                pltpu.async_copy(head_h.at[pl.ds(k * CHUNK, CHUNK)], cb.at[p],
                                 sem_c)
            elif k < 2 * NCHUNK:
                pltpu.async_copy(tail_h.at[pl.ds((k - NCHUNK) * CHUNK, CHUNK)],
                                 cb.at[p], sem_c)

        fire_chunk(0, 0)
        for k in range(2 * NCHUNK):
            p = k % 2
            pltpu.make_async_copy(head_h.at[pl.ds(0, CHUNK)], cb.at[p],
                                  sem_c).wait()
            fire_chunk(1 - p, k + 1)
            base_j = (k * CHUNK) if k < NCHUNK else (B + (k - NCHUNK) * CHUNK)

            def scanv(v, _):
                e = cb[p, pl.ds(v * 16, 16)]
                m = (e >> 15) == wid
                packed = ((e & 32767) << 16) | (base_j + v * 16 + iota)
                cnt = st[0]
                plsc.store_compressed(matchbuf.at[pl.ds(cnt, 16)], packed,
                                      mask=m)
                st[0] = cnt + plsc.all_reduce_population_count(m)[0]
                return 0

            lax.fori_loop(0, CHUNK // 16, scanv, 0)

        cnt = st[0]
        matchbuf[pl.ds(cnt, 16)] = jnp.full((16,), SENT, jnp.int32)
        nv = (cnt >> 4) + 1

        st[1] = 0
        for sb in range(8):
            st[9 + sb] = st[1]

            def sortv(v, _):
                pk = matchbuf[pl.ds(v * 16, 16)]
                m = (pk >> 28) == sb
                cur = st[1]
                plsc.store_compressed(sorted_.at[pl.ds(cur, 16)], pk, mask=m)
                st[1] = cur + plsc.all_reduce_population_count(m)[0]
                return 0

            lax.fori_loop(0, nv, sortv, 0)
        st[17] = st[1]

        nslab = jnp.where(wid == 30, 68, NSLAB)

        def fire_slab(p, b):
            @pl.when((b < nslab) & ((wid < 30) | (b < 66)))
            def _():
                off = wid * NTILE_E + b * W
                pltpu.async_copy(ent_t_h.at[:, pl.ds(off, W)], slab.at[p],
                                 sem_s)

            @pl.when((wid == 30) & (b == 66))
            def _():
                pltpu.async_copy(tailtab_h.at[:, :],
                                 slab.at[p, :, pl.ds(0, 128)], sem_s)

        def wait_slab(p, b):
            @pl.when((b < nslab) & ((wid < 30) | (b < 66)))
            def _():
                pltpu.make_async_copy(ent_t_h.at[:, pl.ds(0, W)],
                                      slab.at[p], sem_s).wait()

            @pl.when((wid == 30) & (b == 66))
            def _():
                pltpu.make_async_copy(tailtab_h.at[:, :],
                                      slab.at[p, :, pl.ds(0, 128)],
                                      sem_s).wait()

        def flush():
            ping = st[3]
            for q in range(2):
                @pl.when(ping == q)
                def _():
                    pltpu.async_copy(gb.at[q], scr_h.at[jbs[q]], sem_w)
                    st[5 + q] = 1
            for q in range(2):
                @pl.when((ping == 1 - q) & (st[5 + q] == 1))
                def _():
                    pltpu.make_async_copy(gb.at[q], scr_h.at[jbs[q]],
                                          sem_w).wait()
                    st[5 + q] = 0
                    reset_jb(jbs[q])
            st[3] = 1 - ping
            st[2] = 0

        def group(p):
            pkv = pend[pl.ds(0, 16)]
            ev = (pkv >> 16) & (W - 1)
            jraw = pkv & 65535
            jrow = jraw + jnp.where(jraw < B, 0, T_OFF - B)
            cur = st[2]
            ping = st[3]
            for q in range(2):
                @pl.when(ping == q)
                def _():
                    jbs[q][pl.ds(cur, 16)] = jrow
            pingv = ping + jnp.zeros((16,), jnp.int32)
            rowv = cur + iota
            for d in range(D):
                dsp = d + jnp.zeros((16,), jnp.int32)
                vals = plsc.load_gather(slab.at[p], [dsp, ev])
                plsc.store_scatter(gb, [pingv, rowv, dsp], vals)
            st[2] = cur + 16

            @pl.when(cur + 16 == FLUSH)
            def _():
                flush()

        def process_slab(p, b):
            sbr = b >> 4
            hi = st[10 + sbr]
            v0 = st[9 + sbr] >> 4
            v1 = (hi + 15) >> 4

            def scanm(v, _):
                pk = sorted_[pl.ds(v * 16, 16)]
                lane = v * 16 + iota
                m = ((pk >> 24) == b) & (lane < hi)
                pc = st[4]
                plsc.store_compressed(pend.at[pl.ds(pc, 16)], pk, mask=m)
                pc2 = pc + plsc.all_reduce_population_count(m)[0]
                st[4] = pc2

                @pl.when(pc2 >= 16)
                def _():
                    group(p)
                    pend[pl.ds(0, 16)] = pend[pl.ds(16, 16)]
                    st[4] = pc2 - 16

                return 0

            lax.fori_loop(v0, v1, scanm, 0)

            @pl.when(st[4] > 0)
            def _():
                pc = st[4]
                pend[pl.ds(pc, 16)] = jnp.full((16,), SENT, jnp.int32)
                group(p)
                st[4] = 0

        fire_slab(0, 0)

        def sweep(b2, _):
            b0 = b2 * 2
            wait_slab(0, b0)
            fire_slab(1, b0 + 1)
            process_slab(0, b0)
            wait_slab(1, b0 + 1)

            @pl.when(b0 + 2 < nslab)
            def _():
                fire_slab(0, b0 + 2)

            process_slab(1, b0 + 1)
            return 0

        lax.fori_loop(0, nslab >> 1, sweep, 0)

        @pl.when(st[2] > 0)
        def _():
            flush()

        for q in range(2):
            @pl.when(st[5 + q] == 1)
            def _():
                pltpu.make_async_copy(gb.at[q], scr_h.at[jbs[q]],
                                      sem_w).wait()


def _bodyB(scr_h, relt_h, relidx_h, out_h,
           rows_h, rows_t, relslab, ridx, outv, semB):
    iota = lax.iota(jnp.int32, 16)
    wid = lax.axis_index("s") * NC + lax.axis_index("c")
    base = wid * 512
    pltpu.sync_copy(relt_h, relslab)
    pltpu.sync_copy(relidx_h.at[pl.ds(base, 512)], ridx.at[pl.ds(0, 512)])

    for ch in range(4):
        rb = base + ch * 128
        pltpu.sync_copy(scr_h.at[pl.ds(rb, 128), :], rows_h)
        pltpu.sync_copy(scr_h.at[pl.ds(T_OFF + rb, 128), :], rows_t)

        def rowgroup(g, _):
            outvec = jnp.zeros((16,), jnp.float32)
            for i in range(16):
                row = g * 16 + i
                ri = ridx[pl.ds(ch * 128 + row, 16)][0]
                riv = ri + jnp.zeros((16,), jnp.int32)
                acc = None
                for c in range(4):
                    rv = plsc.load_gather(relslab, [iota + 16 * c, riv])
                    hv = rows_h[row, pl.ds(c * 16, 16)]
                    tv = rows_t[row, pl.ds(c * 16, 16)]
                    d = jnp.abs(hv + rv - tv)
                    acc = d if acc is None else acc + d
                s = jnp.sum(acc)
                outvec = jnp.where(iota == i, s, outvec)
            outv[pl.ds(ch * 128 + g * 16, 16)] = 0.0 - outvec
            return 0

        lax.fori_loop(0, 8, rowgroup, 0)

    pltpu.sync_copy(outv, out_h.at[pl.ds(base, 512)])


def _make_kernels():
    mesh = plsc.VectorSubcoreMesh(core_axis_name="c", subcore_axis_name="s")
    params = pltpu.CompilerParams(
        needs_layout_passes=False, use_tc_tiling_on_sc=True)
    ka = pl.kernel(
        _bodyA,
        out_type=jax.ShapeDtypeStruct((SCR_ROWS, 128), jnp.float32),
        mesh=mesh,
        compiler_params=params,
        scratch_types=[
            pltpu.VMEM((2, CHUNK), jnp.int32),
            pltpu.VMEM((32800,), jnp.int32),
            pltpu.VMEM((32800,), jnp.int32),
            pltpu.VMEM((48,), jnp.int32),
            pltpu.VMEM((2, D, W), jnp.float32),
            pltpu.VMEM((2, GROWS, 128), jnp.float32),
            pltpu.VMEM((GROWS,), jnp.int32),
            pltpu.VMEM((GROWS,), jnp.int32),
            pltpu.SMEM((32,), jnp.int32),
            pltpu.SemaphoreType.DMA,
            pltpu.SemaphoreType.DMA,
            pltpu.SemaphoreType.DMA,
        ],
    )
    kb = pl.kernel(
        _bodyB,
        out_type=jax.ShapeDtypeStruct((B,), jnp.float32),
        mesh=mesh,
        compiler_params=params,
        scratch_types=[
            pltpu.VMEM((128, 128), jnp.float32),
            pltpu.VMEM((128, 128), jnp.float32),
            pltpu.VMEM((D, 1024), jnp.float32),
            pltpu.VMEM((528,), jnp.int32),
            pltpu.VMEM((512,), jnp.float32),
            pltpu.SemaphoreType.DMA,
        ],
    )
    return ka, kb


_KA, _KB = _make_kernels()


@jax.jit
def _transe(head, rel, tail, ent_embedding, rel_embedding):
    ent_t = ent_embedding.T
    tailtab = jnp.pad(ent_embedding[TAIL_E:], ((0, 64), (0, 0))).T
    rel_t = jnp.pad(rel_embedding, ((0, 24), (0, 0))).T
    scr = _KA(head, tail, ent_t, tailtab)
    return _KB(scr, rel_t, rel)


def kernel(head, rel, tail, ent_embedding, rel_embedding):
    return _transe(head, rel, tail, ent_embedding, rel_embedding).reshape(B, 1)


# speedup vs baseline: 2.8009x; 2.8009x over previous
import jax
import jax.numpy as jnp
from jax import lax
from jax.experimental import pallas as pl
from jax.experimental.pallas import tpu as pltpu
from jax.experimental.pallas import tpu_sc as plsc

E = 1000000
D = 64
B = 16384
NC = 2
NS = 16
NW = NC * NS
RPW = B // NW
CH = 128
NCH = RPW // CH


def _body(head_h, rel_h, tail_h, ent5_h, rel5_h, out_h,
          hidx, ridx, tidx, hrow, rrow, trow, outv, sem):
    iota = lax.iota(jnp.int32, 16)
    wid = lax.axis_index("s") * NC + lax.axis_index("c")
    base = wid * RPW

    for j in range(NCH):
        cbase = base + j * CH
        pltpu.sync_copy(head_h.at[pl.ds(cbase, CH)], hidx.at[j])
        pltpu.sync_copy(rel_h.at[pl.ds(cbase, CH)], ridx.at[j])
        pltpu.sync_copy(tail_h.at[pl.ds(cbase, CH)], tidx.at[j])
        for v in range(CH // 16):
            sl = pl.ds(v * 16, 16)
            hidx[j + NCH, sl] = hidx[j, sl] >> 1
            ridx[j + NCH, sl] = ridx[j, sl] >> 1
            tidx[j + NCH, sl] = tidx[j, sl] >> 1

    def fire(j):
        p = j % 2
        pltpu.async_copy(ent5_h.at[hidx.at[j + NCH]], hrow.at[p], sem)
        pltpu.async_copy(ent5_h.at[tidx.at[j + NCH]], trow.at[p], sem)
        pltpu.async_copy(rel5_h.at[ridx.at[j + NCH]], rrow.at[p], sem)

    def drain(j):
        p = j % 2
        pltpu.make_async_copy(ent5_h.at[hidx.at[j + NCH]], hrow.at[p], sem).wait()
        pltpu.make_async_copy(ent5_h.at[tidx.at[j + NCH]], trow.at[p], sem).wait()
        pltpu.make_async_copy(rel5_h.at[ridx.at[j + NCH]], rrow.at[p], sem).wait()

    fire(0)
    for j in range(NCH):
        p = j % 2
        drain(j)
        if j + 1 < NCH:
            fire(j + 1)

        def group(g, _):
            outvec = jnp.zeros((16,), jnp.float32)
            for i in range(16):
                row = g * 16 + i
                hp = (hidx[j, pl.ds(row, 16)][0] & 1) * D
                rp = (ridx[j, pl.ds(row, 16)][0] & 1) * D
                tp = (tidx[j, pl.ds(row, 16)][0] & 1) * D
                acc = None
                for c in range(D // 16):
                    hv = hrow[p, row, pl.ds(hp + c * 16, 16)]
                    rv = rrow[p, row, pl.ds(rp + c * 16, 16)]
                    tv = trow[p, row, pl.ds(tp + c * 16, 16)]
                    d = jnp.abs(hv + rv - tv)
                    acc = d if acc is None else acc + d
                s = jnp.sum(acc)
                outvec = jnp.where(iota == i, s, outvec)
            outv[pl.ds(j * CH + g * 16, 16)] = 0.0 - outvec
            return 0

        lax.fori_loop(0, CH // 16, group, 0)

    pltpu.sync_copy(outv, out_h.at[pl.ds(base, RPW)])


@jax.jit
def _transe_sc(head, rel, tail, ent_embedding, rel_embedding):
    mesh = plsc.VectorSubcoreMesh(core_axis_name="c", subcore_axis_name="s")
    fn = pl.kernel(
        _body,
        out_type=jax.ShapeDtypeStruct((B,), jnp.float32),
        mesh=mesh,
        compiler_params=pltpu.CompilerParams(
            needs_layout_passes=False, use_tc_tiling_on_sc=True),
        scratch_types=[
            pltpu.VMEM((2 * NCH, CH), jnp.int32),
            pltpu.VMEM((2 * NCH, CH), jnp.int32),
            pltpu.VMEM((2 * NCH, CH), jnp.int32),
            pltpu.VMEM((2, CH, 128), jnp.float32),
            pltpu.VMEM((2, CH, 128), jnp.float32),
            pltpu.VMEM((2, CH, 128), jnp.float32),
            pltpu.VMEM((RPW,), jnp.float32),
            pltpu.SemaphoreType.DMA,
        ],
    )
    ent5 = ent_embedding.reshape(E // 2, 128)
    rel5 = jnp.pad(rel_embedding, ((0, 24), (0, 0))).reshape(512, 128)
    return fn(head, rel, tail, ent5, rel5)


def kernel(head, rel, tail, ent_embedding, rel_embedding):
    return _transe_sc(head, rel, tail, ent_embedding, rel_embedding).reshape(B, 1)
